# baseline (speedup 1.0000x reference)
import functools

import jax
import jax.numpy as jnp
from jax import lax
from jax.experimental import pallas as pl
from jax.experimental.pallas import tpu as pltpu
from jax.experimental.pallas import tpu_sc as plsc

DIM = 512
HID = 2048
E = 8
NBUF = 4
T = 512
L = 16


def _logits_body(x_ref, gw_ref, o_ref):
    o_ref[...] = jax.lax.dot_general(
        gw_ref[...], x_ref[...], (((1,), (1,)), ((), ())),
        preferred_element_type=jnp.float32)


def _make_sc_router():
    info = plsc.get_sparse_core_info()
    nc, ns = info.num_cores, info.num_subcores
    mesh = plsc.VectorSubcoreMesh(core_axis_name="c", subcore_axis_name="s")

    @functools.partial(
        pl.kernel, mesh=mesh,
        out_type=jax.ShapeDtypeStruct((E, T), jnp.float32),
        scratch_types=[
            pltpu.VMEM((E, L), jnp.float32),
            pltpu.VMEM((E, L), jnp.float32),
        ],
    )
    def router(lt_hbm, out_hbm, lv, ov):
        wid = lax.axis_index("s") * nc + lax.axis_index("c")
        base = wid * L
        for e in range(E):
            pltpu.sync_copy(lt_hbm.at[e, pl.ds(base, L)], lv.at[e])
        vs = [lv[e] for e in range(E)]
        m1 = vs[0]
        for e in range(1, E):
            m1 = jnp.maximum(m1, vs[e])
        big = jnp.full((L,), E, jnp.int32)
        i1 = big
        for e in range(E):
            i1 = jnp.minimum(i1, jnp.where(vs[e] == m1, jnp.int32(e), big))
        mvs = [jnp.where(i1 == e, -jnp.inf, vs[e]) for e in range(E)]
        m2 = mvs[0]
        for e in range(1, E):
            m2 = jnp.maximum(m2, mvs[e])
        i2 = big
        for e in range(E):
            i2 = jnp.minimum(i2, jnp.where(mvs[e] == m2, jnp.int32(e), big))
        p1 = 1.0 / (1.0 + jnp.exp(m2 - m1))
        p2 = 1.0 - p1
        zero = jnp.zeros((L,), jnp.float32)
        for e in range(E):
            ov[e, :] = jnp.where(i1 == e, p1, jnp.where(i2 == e, p2, zero))
            pltpu.sync_copy(ov.at[e], out_hbm.at[e, pl.ds(base, L)])

    return router


_sc_router = _make_sc_router()


def _ffn_body(x_ref, wft_ref, w1_hbm, w2_hbm, o_ref, w1buf, w2buf, sems):
    def copy1(e, b):
        return pltpu.make_async_copy(w1_hbm.at[e], w1buf.at[b], sems.at[b, 0])

    def copy2(e, b):
        return pltpu.make_async_copy(w2_hbm.at[e], w2buf.at[b], sems.at[b, 1])

    for e in range(NBUF):
        copy1(e, e).start()
        copy2(e, e).start()

    xb = x_ref[...]
    wf = jnp.transpose(wft_ref[...])

    for e in range(E):
        b = e % NBUF
        copy1(e, b).wait()
        hh = jax.lax.dot_general(
            xb, w1buf[b], (((1,), (1,)), ((), ())),
            preferred_element_type=jnp.float32)
        hh = jnp.maximum(hh, 0.0)
        copy2(e, b).wait()
        y = jax.lax.dot_general(
            hh, w2buf[b], (((1,), (1,)), ((), ())),
            preferred_element_type=jnp.float32)
        contrib = wf[:, e:e + 1] * y
        if e == 0:
            o_ref[...] = contrib
        else:
            o_ref[...] += contrib
        if e + NBUF < E:
            copy1(e + NBUF, b).start()
            copy2(e + NBUF, b).start()


@jax.jit
def kernel(x, gate_w, w1, w2):
    B, N, D = x.shape
    xf = x.reshape(B * N, D)
    logits_t = pl.pallas_call(
        _logits_body,
        in_specs=[
            pl.BlockSpec(memory_space=pltpu.VMEM),
            pl.BlockSpec(memory_space=pltpu.VMEM),
        ],
        out_specs=pl.BlockSpec(memory_space=pltpu.VMEM),
        out_shape=jax.ShapeDtypeStruct((E, B * N), jnp.float32),
    )(xf, gate_w)
    wft = _sc_router(logits_t)
    out = pl.pallas_call(
        _ffn_body,
        in_specs=[
            pl.BlockSpec(memory_space=pltpu.VMEM),
            pl.BlockSpec(memory_space=pltpu.VMEM),
            pl.BlockSpec(memory_space=pl.ANY),
            pl.BlockSpec(memory_space=pl.ANY),
        ],
        out_specs=pl.BlockSpec(memory_space=pltpu.VMEM),
        out_shape=jax.ShapeDtypeStruct((B * N, D), jnp.float32),
        scratch_shapes=[
            pltpu.VMEM((NBUF, HID, DIM), jnp.float32),
            pltpu.VMEM((NBUF, DIM, HID), jnp.float32),
            pltpu.SemaphoreType.DMA((NBUF, 2)),
        ],
    )(xf, wft, w1, w2)
    return out.reshape(B, N, D)


# speedup vs baseline: 1.3011x; 1.3011x over previous
import jax
import jax.numpy as jnp
from jax.experimental import pallas as pl
from jax.experimental.pallas import tpu as pltpu

DIM = 512
HID = 2048
E = 8
NBUF = 3
HH = HID // 2
DH = DIM // 2


def _gate_weights(logits):
    T = logits.shape[0]
    col = jax.lax.broadcasted_iota(jnp.int32, (T, E), 1)
    m1 = jnp.max(logits, axis=1, keepdims=True)
    big = jnp.int32(E)
    idx1 = jnp.min(jnp.where(logits == m1, col, big), axis=1, keepdims=True)
    masked = jnp.where(col == idx1, -jnp.inf, logits)
    m2 = jnp.max(masked, axis=1, keepdims=True)
    idx2 = jnp.min(jnp.where(masked == m2, col, big), axis=1, keepdims=True)
    e2 = jnp.exp(m2 - m1)
    p1 = 1.0 / (1.0 + e2)
    p2 = 1.0 - p1
    return jnp.where(col == idx1, p1, jnp.where(col == idx2, p2, 0.0))


def _moe_body(x_ref, gw_ref, w1_hbm, w2_hbm, o_ref,
              w1a, w1b, w2a, w2b, sems):
    def cps(e, b):
        return (
            pltpu.make_async_copy(w1_hbm.at[e, pl.ds(0, HH)], w1a.at[b],
                                  sems.at[b, 0]),
            pltpu.make_async_copy(w1_hbm.at[e, pl.ds(HH, HH)], w1b.at[b],
                                  sems.at[b, 1]),
            pltpu.make_async_copy(w2_hbm.at[e, pl.ds(0, DH)], w2a.at[b],
                                  sems.at[b, 2]),
            pltpu.make_async_copy(w2_hbm.at[e, pl.ds(DH, DH)], w2b.at[b],
                                  sems.at[b, 3]),
        )

    for e in range(NBUF):
        for c in cps(e, e):
            c.start()

    xb = x_ref[...]
    logits = jax.lax.dot_general(
        xb, gw_ref[...], (((1,), (1,)), ((), ())),
        preferred_element_type=jnp.float32)
    wf = _gate_weights(logits)
    ctr11 = (((1,), (1,)), ((), ()))

    for e in range(E):
        b = e % NBUF
        c1a, c1b, c2a, c2b = cps(e, b)
        we = wf[:, e:e + 1]
        c1a.wait()
        ha = jnp.maximum(jax.lax.dot_general(
            xb, w1a[b], ctr11, preferred_element_type=jnp.float32), 0.0)
        c1b.wait()
        hb = jnp.maximum(jax.lax.dot_general(
            xb, w1b[b], ctr11, preferred_element_type=jnp.float32), 0.0)
        c2a.wait()
        ya = (jax.lax.dot_general(ha, w2a[b][:, :HH], ctr11,
                                  preferred_element_type=jnp.float32)
              + jax.lax.dot_general(hb, w2a[b][:, HH:], ctr11,
                                    preferred_element_type=jnp.float32))
        if e == 0:
            o_ref[:, :DH] = we * ya
        else:
            o_ref[:, :DH] += we * ya
        c2b.wait()
        yb = (jax.lax.dot_general(ha, w2b[b][:, :HH], ctr11,
                                  preferred_element_type=jnp.float32)
              + jax.lax.dot_general(hb, w2b[b][:, HH:], ctr11,
                                    preferred_element_type=jnp.float32))
        if e == 0:
            o_ref[:, DH:] = we * yb
        else:
            o_ref[:, DH:] += we * yb
        if e + NBUF < E:
            for c in cps(e + NBUF, b):
                c.start()


@jax.jit
def kernel(x, gate_w, w1, w2):
    B, N, D = x.shape
    T = B * N
    out = pl.pallas_call(
        _moe_body,
        in_specs=[
            pl.BlockSpec(memory_space=pltpu.VMEM),
            pl.BlockSpec(memory_space=pltpu.VMEM),
            pl.BlockSpec(memory_space=pl.ANY),
            pl.BlockSpec(memory_space=pl.ANY),
        ],
        out_specs=pl.BlockSpec(memory_space=pltpu.VMEM),
        out_shape=jax.ShapeDtypeStruct((T, D), jnp.float32),
        scratch_shapes=[
            pltpu.VMEM((NBUF, HH, DIM), jnp.float32),
            pltpu.VMEM((NBUF, HH, DIM), jnp.float32),
            pltpu.VMEM((NBUF, DH, HID), jnp.float32),
            pltpu.VMEM((NBUF, DH, HID), jnp.float32),
            pltpu.SemaphoreType.DMA((NBUF, 4)),
        ],
    )(x.reshape(T, D), gate_w, w1, w2)
    return out.reshape(B, N, D)


# speedup vs baseline: 1.8350x; 1.4103x over previous
import jax
import jax.numpy as jnp
from jax.experimental import pallas as pl
from jax.experimental.pallas import tpu as pltpu

DIM = 512
HID = 2048
E = 8
NBUF = 3


def _gate_weights(logits):
    T = logits.shape[0]
    col = jax.lax.broadcasted_iota(jnp.int32, (T, E), 1)
    m1 = jnp.max(logits, axis=1, keepdims=True)
    big = jnp.int32(E)
    idx1 = jnp.min(jnp.where(logits == m1, col, big), axis=1, keepdims=True)
    masked = jnp.where(col == idx1, -jnp.inf, logits)
    m2 = jnp.max(masked, axis=1, keepdims=True)
    idx2 = jnp.min(jnp.where(masked == m2, col, big), axis=1, keepdims=True)
    e2 = jnp.exp(m2 - m1)
    p1 = 1.0 / (1.0 + e2)
    p2 = 1.0 - p1
    return jnp.where(col == idx1, p1, jnp.where(col == idx2, p2, 0.0))


def _moe_body(x_ref, gw_ref, w1_hbm, w2_hbm, o_ref, w1buf, w2buf, sems):
    def copy1(e, b):
        return pltpu.make_async_copy(w1_hbm.at[e], w1buf.at[b], sems.at[b, 0])

    def copy2(e, b):
        return pltpu.make_async_copy(w2_hbm.at[e], w2buf.at[b], sems.at[b, 1])

    for e in range(NBUF):
        copy1(e, e).start()
        copy2(e, e).start()

    xb = x_ref[...]
    logits = jax.lax.dot_general(
        xb, gw_ref[...], (((1,), (1,)), ((), ())),
        preferred_element_type=jnp.float32)
    wf = _gate_weights(logits)

    for e in range(E):
        b = e % NBUF
        copy1(e, b).wait()
        copy2(e, b).wait()
        hh = jax.lax.dot_general(
            xb, w1buf[b], (((1,), (1,)), ((), ())),
            preferred_element_type=jnp.float32)
        hh = jnp.maximum(hh, 0.0)
        y = jax.lax.dot_general(
            hh, w2buf[b], (((1,), (1,)), ((), ())),
            preferred_element_type=jnp.float32)
        contrib = wf[:, e:e + 1] * y
        if e == 0:
            o_ref[...] = contrib
        else:
            o_ref[...] += contrib
        if e + NBUF < E:
            copy1(e + NBUF, b).start()
            copy2(e + NBUF, b).start()


@jax.jit
def kernel(x, gate_w, w1, w2):
    B, N, D = x.shape
    T = B * N
    out = pl.pallas_call(
        _moe_body,
        in_specs=[
            pl.BlockSpec(memory_space=pltpu.VMEM),
            pl.BlockSpec(memory_space=pltpu.VMEM),
            pl.BlockSpec(memory_space=pl.ANY),
            pl.BlockSpec(memory_space=pl.ANY),
        ],
        out_specs=pl.BlockSpec(memory_space=pltpu.VMEM),
        out_shape=jax.ShapeDtypeStruct((T, D), jnp.float32),
        scratch_shapes=[
            pltpu.VMEM((NBUF, HID, DIM), jnp.float32),
            pltpu.VMEM((NBUF, DIM, HID), jnp.float32),
            pltpu.SemaphoreType.DMA((NBUF, 2)),
        ],
    )(x.reshape(T, D), gate_w, w1, w2)
    return out.reshape(B, N, D)
